# baseline (speedup 1.0000x reference)
import functools

import numpy as np

import jax
import jax.numpy as jnp
from jax import lax
from jax.experimental import pallas as pl
from jax.experimental.pallas import tpu as pltpu
from jax.experimental.pallas import tpu_sc as plsc

N = 10000
E = 320000
F_IN = 128
HID = 128
HX = 64
C = 40

BLK = 5000
NW = 32
EW = E // NW
L = 16


def _tc_body(x_ref, ym_ref, w1_ref, b1_ref, w2_ref, b2_ref,
             wx_ref, bx_ref, wabx_ref, waby_ref, peb_ref,
             ylp_ref, ab_ref):
    x = x_ref[...]
    h = jnp.maximum(x @ w1_ref[...] + b1_ref[...], 0.0)
    logits = h @ w2_ref[...] + b2_ref[...]
    mx = jnp.max(logits, axis=-1, keepdims=True)
    ex = jnp.exp(logits - mx)
    s = jnp.sum(ex, axis=-1, keepdims=True)
    lse = jnp.log(s) + mx
    ylp = logits - lse
    ylp_ref[...] = ylp
    yprob = ex * (1.0 / s)
    yf = ym_ref[:, 0:1]
    m = ym_ref[:, 1:2]
    onehot = (lax.broadcasted_iota(jnp.int32, (BLK, C), 1).astype(jnp.float32)
              == yf).astype(jnp.float32)
    ypeff = onehot * m + yprob * (1.0 - m)
    xe = jnp.maximum(x @ wx_ref[...] + bx_ref[...], 0.0)
    ab_ref[...] = xe @ wabx_ref[...] + ypeff @ waby_ref[...] + peb_ref[...]


def _node_stage(x, ym, fc1_w, fc1_b2, fc2_w, fc2_b2, xenc_w, xenc_b2,
                wab_x, wab_y, peb2):
    grid = (N // BLK,)
    full = lambda shape: pl.BlockSpec(shape, lambda i: (0, 0))
    blk = lambda w: pl.BlockSpec((BLK, w), lambda i: (i, 0))
    return pl.pallas_call(
        _tc_body,
        grid=grid,
        in_specs=[
            blk(F_IN), blk(2),
            full((F_IN, HID)), full((1, HID)),
            full((HID, C)), full((1, C)),
            full((F_IN, HX)), full((1, HX)),
            full((HX, 2)), full((C, 2)), full((1, 2)),
        ],
        out_specs=[blk(C), blk(2)],
        out_shape=[
            jax.ShapeDtypeStruct((N, C), jnp.float32),
            jax.ShapeDtypeStruct((N, 2), jnp.float32),
        ],
    )(x, ym, fc1_w, fc1_b2, fc2_w, fc2_b2, xenc_w, xenc_b2,
      wab_x, wab_y, peb2)


def _sc_edge_body(ab_hbm, ei_hbm, nsrc_hbm, ndst_hbm, pos_out, neg_out,
                  ab_v, src_v, dst_v, nsrc_v, ndst_v, out_v, out2_v, sem):
    wid = lax.axis_index("s") * 2 + lax.axis_index("c")
    base = wid * EW
    copies = [
        pltpu.async_copy(ab_hbm, ab_v, sem),
        pltpu.async_copy(ei_hbm.at[pl.ds(base, EW)], src_v, sem),
        pltpu.async_copy(ei_hbm.at[pl.ds(E + base, EW)], dst_v, sem),
        pltpu.async_copy(nsrc_hbm.at[pl.ds(base, EW)], nsrc_v, sem),
        pltpu.async_copy(ndst_hbm.at[pl.ds(base, EW)], ndst_v, sem),
    ]
    for c in copies:
        c.wait()

    @plsc.parallel_loop(0, EW, step=L, unroll=4)
    def _pos(off):
        idx_s = src_v[pl.ds(off, L)]
        idx_d = dst_v[pl.ds(off, L)]
        va = plsc.load_gather(ab_v, [idx_s + idx_s])
        vb = plsc.load_gather(ab_v, [idx_d + idx_d + 1])
        out_v[pl.ds(off, L)] = va + vb

    cp = pltpu.async_copy(out_v, pos_out.at[0, pl.ds(base, EW)], sem)

    @plsc.parallel_loop(0, EW, step=L, unroll=4)
    def _neg(off):
        idx_s = nsrc_v[pl.ds(off, L)]
        idx_d = ndst_v[pl.ds(off, L)]
        va = plsc.load_gather(ab_v, [idx_s + idx_s])
        vb = plsc.load_gather(ab_v, [idx_d + idx_d + 1])
        out2_v[pl.ds(off, L)] = va + vb

    cp.wait()
    pltpu.sync_copy(out2_v, neg_out.at[0, pl.ds(base, EW)])


@functools.cache
def _make_sc_edge():
    return pl.kernel(
        _sc_edge_body,
        out_type=(
            jax.ShapeDtypeStruct((1, E), jnp.float32),
            jax.ShapeDtypeStruct((1, E), jnp.float32),
        ),
        mesh=plsc.VectorSubcoreMesh(core_axis_name="c", subcore_axis_name="s",
                                    num_cores=2, num_subcores=16),
        scratch_types=[
            pltpu.VMEM((2 * N,), jnp.float32),
            pltpu.VMEM((EW,), jnp.int32),
            pltpu.VMEM((EW,), jnp.int32),
            pltpu.VMEM((EW,), jnp.int32),
            pltpu.VMEM((EW,), jnp.int32),
            pltpu.VMEM((EW,), jnp.float32),
            pltpu.VMEM((EW,), jnp.float32),
            pltpu.SemaphoreType.DMA,
        ],
        compiler_params=pltpu.CompilerParams(needs_layout_passes=False,
                                             use_tc_tiling_on_sc=False),
    )


def _tf2x32(k1, k2, x1, x2):
    rot = [np.uint32(r) for r in (13, 15, 26, 6, 17, 29, 16, 24)]

    def rotl(v, r):
        return (v << r) | (v >> np.uint32(32 - int(r)))

    def rounds(x0, x1, rs):
        for r in rs:
            x0 = x0 + x1
            x1 = rotl(x1, r)
            x1 = x1 ^ x0
        return x0, x1

    ks = [k1, k2, k1 ^ k2 ^ np.uint32(0x1BD11BDA)]
    x0, x1 = x1 + ks[0], x2 + ks[1]
    x0, x1 = rounds(x0, x1, rot[0:4])
    x0, x1 = x0 + ks[1], x1 + ks[2] + np.uint32(1)
    x0, x1 = rounds(x0, x1, rot[4:8])
    x0, x1 = x0 + ks[2], x1 + ks[0] + np.uint32(2)
    x0, x1 = rounds(x0, x1, rot[0:4])
    x0, x1 = x0 + ks[0], x1 + ks[1] + np.uint32(3)
    x0, x1 = rounds(x0, x1, rot[4:8])
    x0, x1 = x0 + ks[1], x1 + ks[2] + np.uint32(4)
    x0, x1 = rounds(x0, x1, rot[0:4])
    x0, x1 = x0 + ks[2], x1 + ks[0] + np.uint32(5)
    return x0, x1


def _random_bits32(k, size):
    idx = np.arange(size, dtype=np.uint64)
    c1 = (idx >> np.uint64(32)).astype(np.uint32)
    c2 = (idx & np.uint64(0xFFFFFFFF)).astype(np.uint32)
    b1, b2 = _tf2x32(k[0], k[1], c1, c2)
    return b1 ^ b2


@functools.cache
def _neg_edges():
    with np.errstate(over="ignore"):
        c1 = np.zeros(2, np.uint32)
        c2 = np.arange(2, dtype=np.uint32)
        b1, b2 = _tf2x32(np.uint32(0), np.uint32(42), c1, c2)
        hi = _random_bits32((b1[0], b2[0]), 2 * E)
        lo = _random_bits32((b1[1], b2[1]), 2 * E)
        span = np.uint32(N)
        mult = np.uint32(((2 ** 16) % N) ** 2 % N)
        off = ((hi % span) * mult + (lo % span)) % span
    arr = off.astype(np.int32).reshape(2, E)
    return np.ascontiguousarray(arr[0]), np.ascontiguousarray(arr[1])


def kernel(x, edge_index, y, train_mask, fc1_w, fc1_b, fc2_w, fc2_b,
           xenc_w, xenc_b, pe_w, pe_b):
    ym = jnp.stack([y.astype(jnp.float32),
                    train_mask.astype(jnp.float32)], axis=1)
    wab_x = jnp.concatenate([pe_w[0:HX], pe_w[HX:2 * HX]], axis=1)
    wab_y = jnp.concatenate([pe_w[2 * HX:2 * HX + C],
                             pe_w[2 * HX + C:]], axis=1)
    peb2 = jnp.concatenate([pe_b.reshape(1, 1),
                            jnp.zeros((1, 1), jnp.float32)], axis=1)
    ylp, ab2 = _node_stage(
        x, ym, fc1_w, fc1_b.reshape(1, HID), fc2_w, fc2_b.reshape(1, C),
        xenc_w, xenc_b.reshape(1, HX), wab_x, wab_y, peb2)
    ab_flat = ab2.reshape(2 * N)

    nsrc, ndst = _neg_edges()
    e_pos, e_neg = _make_sc_edge()(ab_flat, edge_index.reshape(2 * E),
                                   jnp.asarray(nsrc), jnp.asarray(ndst))
    return (e_pos.reshape(E, 1), e_neg.reshape(E, 1), ylp)


# speedup vs baseline: 1.0015x; 1.0015x over previous
import functools

import numpy as np

import jax
import jax.numpy as jnp
from jax import lax
from jax.experimental import pallas as pl
from jax.experimental.pallas import tpu as pltpu
from jax.experimental.pallas import tpu_sc as plsc

N = 10000
E = 320000
F_IN = 128
HID = 128
HX = 64
C = 40

BLK = 5000
NW = 32
EW = E // NW
L = 16


def _tc_body(x_ref, ym_ref, w1_ref, b1_ref, w2_ref, b2_ref,
             wx_ref, bx_ref, wabx_ref, waby_ref, peb_ref,
             ylp_ref, ab_ref):
    x = x_ref[...]
    h = jnp.maximum(x @ w1_ref[...] + b1_ref[...], 0.0)
    logits = h @ w2_ref[...] + b2_ref[...]
    mx = jnp.max(logits, axis=-1, keepdims=True)
    ex = jnp.exp(logits - mx)
    s = jnp.sum(ex, axis=-1, keepdims=True)
    lse = jnp.log(s) + mx
    ylp = logits - lse
    ylp_ref[...] = ylp
    yprob = ex * (1.0 / s)
    yf = ym_ref[:, 0:1]
    m = ym_ref[:, 1:2]
    onehot = (lax.broadcasted_iota(jnp.int32, (BLK, C), 1).astype(jnp.float32)
              == yf).astype(jnp.float32)
    ypeff = onehot * m + yprob * (1.0 - m)
    xe = jnp.maximum(x @ wx_ref[...] + bx_ref[...], 0.0)
    ab_ref[...] = xe @ wabx_ref[...] + ypeff @ waby_ref[...] + peb_ref[...]


def _node_stage(x, ym, fc1_w, fc1_b2, fc2_w, fc2_b2, xenc_w, xenc_b2,
                wab_x, wab_y, peb2):
    grid = (N // BLK,)
    full = lambda shape: pl.BlockSpec(shape, lambda i: (0, 0))
    blk = lambda w: pl.BlockSpec((BLK, w), lambda i: (i, 0))
    return pl.pallas_call(
        _tc_body,
        grid=grid,
        in_specs=[
            blk(F_IN), blk(2),
            full((F_IN, HID)), full((1, HID)),
            full((HID, C)), full((1, C)),
            full((F_IN, HX)), full((1, HX)),
            full((HX, 2)), full((C, 2)), full((1, 2)),
        ],
        out_specs=[blk(C), blk(2)],
        out_shape=[
            jax.ShapeDtypeStruct((N, C), jnp.float32),
            jax.ShapeDtypeStruct((N, 2), jnp.float32),
        ],
    )(x, ym, fc1_w, fc1_b2, fc2_w, fc2_b2, xenc_w, xenc_b2,
      wab_x, wab_y, peb2)


def _sc_edge_body(ab_hbm, ei_hbm, nsrc_hbm, ndst_hbm, pos_out, neg_out,
                  ab_v, src_v, dst_v, nsrc_v, ndst_v, out_v, out2_v,
                  sem, sem2):
    wid = lax.axis_index("s") * 2 + lax.axis_index("c")
    base = wid * EW
    pos_copies = [
        pltpu.async_copy(ab_hbm, ab_v, sem),
        pltpu.async_copy(ei_hbm.at[pl.ds(base, EW)], src_v, sem),
        pltpu.async_copy(ei_hbm.at[pl.ds(E + base, EW)], dst_v, sem),
    ]
    neg_copies = [
        pltpu.async_copy(nsrc_hbm.at[pl.ds(base, EW)], nsrc_v, sem2),
        pltpu.async_copy(ndst_hbm.at[pl.ds(base, EW)], ndst_v, sem2),
    ]
    for c in pos_copies:
        c.wait()

    @plsc.parallel_loop(0, EW, step=L, unroll=4)
    def _pos(off):
        idx_s = src_v[pl.ds(off, L)]
        idx_d = dst_v[pl.ds(off, L)]
        va = plsc.load_gather(ab_v, [idx_s + idx_s])
        vb = plsc.load_gather(ab_v, [idx_d + idx_d + 1])
        out_v[pl.ds(off, L)] = va + vb

    cp = pltpu.async_copy(out_v, pos_out.at[0, pl.ds(base, EW)], sem)
    for c in neg_copies:
        c.wait()

    @plsc.parallel_loop(0, EW, step=L, unroll=4)
    def _neg(off):
        idx_s = nsrc_v[pl.ds(off, L)]
        idx_d = ndst_v[pl.ds(off, L)]
        va = plsc.load_gather(ab_v, [idx_s + idx_s])
        vb = plsc.load_gather(ab_v, [idx_d + idx_d + 1])
        out2_v[pl.ds(off, L)] = va + vb

    cp.wait()
    pltpu.sync_copy(out2_v, neg_out.at[0, pl.ds(base, EW)])


@functools.cache
def _make_sc_edge():
    return pl.kernel(
        _sc_edge_body,
        out_type=(
            jax.ShapeDtypeStruct((1, E), jnp.float32),
            jax.ShapeDtypeStruct((1, E), jnp.float32),
        ),
        mesh=plsc.VectorSubcoreMesh(core_axis_name="c", subcore_axis_name="s",
                                    num_cores=2, num_subcores=16),
        scratch_types=[
            pltpu.VMEM((2 * N,), jnp.float32),
            pltpu.VMEM((EW,), jnp.int32),
            pltpu.VMEM((EW,), jnp.int32),
            pltpu.VMEM((EW,), jnp.int32),
            pltpu.VMEM((EW,), jnp.int32),
            pltpu.VMEM((EW,), jnp.float32),
            pltpu.VMEM((EW,), jnp.float32),
            pltpu.SemaphoreType.DMA,
            pltpu.SemaphoreType.DMA,
        ],
        compiler_params=pltpu.CompilerParams(needs_layout_passes=False,
                                             use_tc_tiling_on_sc=False),
    )


def _tf2x32(k1, k2, x1, x2):
    rot = [np.uint32(r) for r in (13, 15, 26, 6, 17, 29, 16, 24)]

    def rotl(v, r):
        return (v << r) | (v >> np.uint32(32 - int(r)))

    def rounds(x0, x1, rs):
        for r in rs:
            x0 = x0 + x1
            x1 = rotl(x1, r)
            x1 = x1 ^ x0
        return x0, x1

    ks = [k1, k2, k1 ^ k2 ^ np.uint32(0x1BD11BDA)]
    x0, x1 = x1 + ks[0], x2 + ks[1]
    x0, x1 = rounds(x0, x1, rot[0:4])
    x0, x1 = x0 + ks[1], x1 + ks[2] + np.uint32(1)
    x0, x1 = rounds(x0, x1, rot[4:8])
    x0, x1 = x0 + ks[2], x1 + ks[0] + np.uint32(2)
    x0, x1 = rounds(x0, x1, rot[0:4])
    x0, x1 = x0 + ks[0], x1 + ks[1] + np.uint32(3)
    x0, x1 = rounds(x0, x1, rot[4:8])
    x0, x1 = x0 + ks[1], x1 + ks[2] + np.uint32(4)
    x0, x1 = rounds(x0, x1, rot[0:4])
    x0, x1 = x0 + ks[2], x1 + ks[0] + np.uint32(5)
    return x0, x1


def _random_bits32(k, size):
    idx = np.arange(size, dtype=np.uint64)
    c1 = (idx >> np.uint64(32)).astype(np.uint32)
    c2 = (idx & np.uint64(0xFFFFFFFF)).astype(np.uint32)
    b1, b2 = _tf2x32(k[0], k[1], c1, c2)
    return b1 ^ b2


@functools.cache
def _neg_edges():
    with np.errstate(over="ignore"):
        c1 = np.zeros(2, np.uint32)
        c2 = np.arange(2, dtype=np.uint32)
        b1, b2 = _tf2x32(np.uint32(0), np.uint32(42), c1, c2)
        hi = _random_bits32((b1[0], b2[0]), 2 * E)
        lo = _random_bits32((b1[1], b2[1]), 2 * E)
        span = np.uint32(N)
        mult = np.uint32(((2 ** 16) % N) ** 2 % N)
        off = ((hi % span) * mult + (lo % span)) % span
    arr = off.astype(np.int32).reshape(2, E)
    return np.ascontiguousarray(arr[0]), np.ascontiguousarray(arr[1])


def kernel(x, edge_index, y, train_mask, fc1_w, fc1_b, fc2_w, fc2_b,
           xenc_w, xenc_b, pe_w, pe_b):
    ym = jnp.stack([y.astype(jnp.float32),
                    train_mask.astype(jnp.float32)], axis=1)
    wab_x = jnp.concatenate([pe_w[0:HX], pe_w[HX:2 * HX]], axis=1)
    wab_y = jnp.concatenate([pe_w[2 * HX:2 * HX + C],
                             pe_w[2 * HX + C:]], axis=1)
    peb2 = jnp.concatenate([pe_b.reshape(1, 1),
                            jnp.zeros((1, 1), jnp.float32)], axis=1)
    ylp, ab2 = _node_stage(
        x, ym, fc1_w, fc1_b.reshape(1, HID), fc2_w, fc2_b.reshape(1, C),
        xenc_w, xenc_b.reshape(1, HX), wab_x, wab_y, peb2)
    ab_flat = ab2.reshape(2 * N)

    nsrc, ndst = _neg_edges()
    e_pos, e_neg = _make_sc_edge()(ab_flat, edge_index.reshape(2 * E),
                                   jnp.asarray(nsrc), jnp.asarray(ndst))
    return (e_pos.reshape(E, 1), e_neg.reshape(E, 1), ylp)


# speedup vs baseline: 1.0334x; 1.0318x over previous
import functools

import numpy as np

import jax
import jax.numpy as jnp
from jax import lax
from jax.experimental import pallas as pl
from jax.experimental.pallas import tpu as pltpu
from jax.experimental.pallas import tpu_sc as plsc

N = 10000
E = 320000
F_IN = 128
HID = 128
HX = 64
C = 40

BLK = 5000
NW = 32
EW = E // NW
L = 16


def _tc_body(x_ref, ym_ref, w1_ref, b1_ref, w2_ref, b2_ref,
             wx_ref, bx_ref, pew_ref, peb_ref,
             ylp_ref, ab_ref):
    x = x_ref[...]
    h = jnp.maximum(x @ w1_ref[...] + b1_ref[...], 0.0)
    logits = h @ w2_ref[...] + b2_ref[...]
    mx = jnp.max(logits, axis=-1, keepdims=True)
    ex = jnp.exp(logits - mx)
    s = jnp.sum(ex, axis=-1, keepdims=True)
    lse = jnp.log(s) + mx
    ylp = logits - lse
    ylp_ref[...] = ylp
    yprob = ex * (1.0 / s)
    yf = ym_ref[:, 0:1]
    m = ym_ref[:, 1:2]
    onehot = (lax.broadcasted_iota(jnp.int32, (BLK, C), 1).astype(jnp.float32)
              == yf).astype(jnp.float32)
    ypeff = onehot * m + yprob * (1.0 - m)
    xe = jnp.maximum(x @ wx_ref[...] + bx_ref[...], 0.0)
    pw = pew_ref[...]
    wabx = jnp.concatenate([pw[0:HX], pw[HX:2 * HX]], axis=1)
    waby = jnp.concatenate([pw[2 * HX:2 * HX + C], pw[2 * HX + C:]], axis=1)
    peb = jnp.concatenate([peb_ref[...], jnp.zeros((1, 1), jnp.float32)],
                          axis=1)
    ab_ref[...] = xe @ wabx + ypeff @ waby + peb


def _node_stage(x, ym, fc1_w, fc1_b2, fc2_w, fc2_b2, xenc_w, xenc_b2,
                pe_w, pe_b2):
    grid = (N // BLK,)
    full = lambda shape: pl.BlockSpec(shape, lambda i: (0, 0))
    blk = lambda w: pl.BlockSpec((BLK, w), lambda i: (i, 0))
    return pl.pallas_call(
        _tc_body,
        grid=grid,
        in_specs=[
            blk(F_IN), blk(2),
            full((F_IN, HID)), full((1, HID)),
            full((HID, C)), full((1, C)),
            full((F_IN, HX)), full((1, HX)),
            full((2 * (HX + C), 1)), full((1, 1)),
        ],
        out_specs=[blk(C), blk(2)],
        out_shape=[
            jax.ShapeDtypeStruct((N, C), jnp.float32),
            jax.ShapeDtypeStruct((N, 2), jnp.float32),
        ],
    )(x, ym, fc1_w, fc1_b2, fc2_w, fc2_b2, xenc_w, xenc_b2,
      pe_w, pe_b2)


def _sc_edge_body(ab_hbm, ei_hbm, nsrc_hbm, ndst_hbm, pos_out, neg_out,
                  ab_v, src_v, dst_v, nsrc_v, ndst_v, out_v, out2_v,
                  sem, sem2):
    wid = lax.axis_index("s") * 2 + lax.axis_index("c")
    base = wid * EW
    pos_copies = [
        pltpu.async_copy(ab_hbm, ab_v, sem),
        pltpu.async_copy(ei_hbm.at[pl.ds(base, EW)], src_v, sem),
        pltpu.async_copy(ei_hbm.at[pl.ds(E + base, EW)], dst_v, sem),
    ]
    neg_copies = [
        pltpu.async_copy(nsrc_hbm.at[pl.ds(base, EW)], nsrc_v, sem2),
        pltpu.async_copy(ndst_hbm.at[pl.ds(base, EW)], ndst_v, sem2),
    ]
    for c in pos_copies:
        c.wait()

    @plsc.parallel_loop(0, EW, step=L, unroll=4)
    def _pos(off):
        idx_s = src_v[pl.ds(off, L)]
        idx_d = dst_v[pl.ds(off, L)]
        va = plsc.load_gather(ab_v, [idx_s + idx_s])
        vb = plsc.load_gather(ab_v, [idx_d + idx_d + 1])
        out_v[pl.ds(off, L)] = va + vb

    cp = pltpu.async_copy(out_v, pos_out.at[0, pl.ds(base, EW)], sem)
    for c in neg_copies:
        c.wait()

    @plsc.parallel_loop(0, EW, step=L, unroll=4)
    def _neg(off):
        idx_s = nsrc_v[pl.ds(off, L)]
        idx_d = ndst_v[pl.ds(off, L)]
        va = plsc.load_gather(ab_v, [idx_s + idx_s])
        vb = plsc.load_gather(ab_v, [idx_d + idx_d + 1])
        out2_v[pl.ds(off, L)] = va + vb

    cp.wait()
    pltpu.sync_copy(out2_v, neg_out.at[0, pl.ds(base, EW)])


@functools.cache
def _make_sc_edge():
    return pl.kernel(
        _sc_edge_body,
        out_type=(
            jax.ShapeDtypeStruct((1, E), jnp.float32),
            jax.ShapeDtypeStruct((1, E), jnp.float32),
        ),
        mesh=plsc.VectorSubcoreMesh(core_axis_name="c", subcore_axis_name="s",
                                    num_cores=2, num_subcores=16),
        scratch_types=[
            pltpu.VMEM((2 * N,), jnp.float32),
            pltpu.VMEM((EW,), jnp.int32),
            pltpu.VMEM((EW,), jnp.int32),
            pltpu.VMEM((EW,), jnp.int32),
            pltpu.VMEM((EW,), jnp.int32),
            pltpu.VMEM((EW,), jnp.float32),
            pltpu.VMEM((EW,), jnp.float32),
            pltpu.SemaphoreType.DMA,
            pltpu.SemaphoreType.DMA,
        ],
        compiler_params=pltpu.CompilerParams(needs_layout_passes=False,
                                             use_tc_tiling_on_sc=False),
    )


def _tf2x32(k1, k2, x1, x2):
    rot = [np.uint32(r) for r in (13, 15, 26, 6, 17, 29, 16, 24)]

    def rotl(v, r):
        return (v << r) | (v >> np.uint32(32 - int(r)))

    def rounds(x0, x1, rs):
        for r in rs:
            x0 = x0 + x1
            x1 = rotl(x1, r)
            x1 = x1 ^ x0
        return x0, x1

    ks = [k1, k2, k1 ^ k2 ^ np.uint32(0x1BD11BDA)]
    x0, x1 = x1 + ks[0], x2 + ks[1]
    x0, x1 = rounds(x0, x1, rot[0:4])
    x0, x1 = x0 + ks[1], x1 + ks[2] + np.uint32(1)
    x0, x1 = rounds(x0, x1, rot[4:8])
    x0, x1 = x0 + ks[2], x1 + ks[0] + np.uint32(2)
    x0, x1 = rounds(x0, x1, rot[0:4])
    x0, x1 = x0 + ks[0], x1 + ks[1] + np.uint32(3)
    x0, x1 = rounds(x0, x1, rot[4:8])
    x0, x1 = x0 + ks[1], x1 + ks[2] + np.uint32(4)
    x0, x1 = rounds(x0, x1, rot[0:4])
    x0, x1 = x0 + ks[2], x1 + ks[0] + np.uint32(5)
    return x0, x1


def _random_bits32(k, size):
    idx = np.arange(size, dtype=np.uint64)
    c1 = (idx >> np.uint64(32)).astype(np.uint32)
    c2 = (idx & np.uint64(0xFFFFFFFF)).astype(np.uint32)
    b1, b2 = _tf2x32(k[0], k[1], c1, c2)
    return b1 ^ b2


@functools.cache
def _neg_edges():
    with np.errstate(over="ignore"):
        c1 = np.zeros(2, np.uint32)
        c2 = np.arange(2, dtype=np.uint32)
        b1, b2 = _tf2x32(np.uint32(0), np.uint32(42), c1, c2)
        hi = _random_bits32((b1[0], b2[0]), 2 * E)
        lo = _random_bits32((b1[1], b2[1]), 2 * E)
        span = np.uint32(N)
        mult = np.uint32(((2 ** 16) % N) ** 2 % N)
        off = ((hi % span) * mult + (lo % span)) % span
    arr = off.astype(np.int32).reshape(2, E)
    return np.ascontiguousarray(arr[0]), np.ascontiguousarray(arr[1])


def kernel(x, edge_index, y, train_mask, fc1_w, fc1_b, fc2_w, fc2_b,
           xenc_w, xenc_b, pe_w, pe_b):
    ym = jnp.stack([y.astype(jnp.float32),
                    train_mask.astype(jnp.float32)], axis=1)
    ylp, ab2 = _node_stage(
        x, ym, fc1_w, fc1_b.reshape(1, HID), fc2_w, fc2_b.reshape(1, C),
        xenc_w, xenc_b.reshape(1, HX), pe_w, pe_b.reshape(1, 1))
    ab_flat = ab2.reshape(2 * N)

    nsrc, ndst = _neg_edges()
    e_pos, e_neg = _make_sc_edge()(ab_flat, edge_index.reshape(2 * E),
                                   jnp.asarray(nsrc), jnp.asarray(ndst))
    return (e_pos.reshape(E, 1), e_neg.reshape(E, 1), ylp)
